# baseline (speedup 1.0000x reference)
import functools

import jax
import jax.numpy as jnp
from jax import lax
from jax.experimental import pallas as pl
from jax.experimental.pallas import tpu as pltpu
from jax.experimental.pallas import tpu_sc as plsc

VOCAB = 100000
EMBED = 64
CONTEXT = 200
HIDDEN = 128

CHUNK = 8192
NCHUNK = -(-VOCAB // CHUNK)
VPAD = NCHUNK * CHUNK

NC, NS = 2, 16
NW = NC * NS
LANES = 16
B_PAD = NW * LANES
B_PER_W = B_PAD // NW
SUPER = 12500


@functools.cache
def _gather_sc():
    @functools.partial(
        pl.kernel,
        mesh=plsc.VectorSubcoreMesh(core_axis_name="c", subcore_axis_name="s"),
        out_type=jax.ShapeDtypeStruct((B_PAD, EMBED), jnp.float32),
        scratch_types=[
            pltpu.VMEM((B_PER_W,), jnp.int32),
            pltpu.VMEM((B_PER_W, EMBED), jnp.float32),
            pltpu.SemaphoreType.DMA,
        ],
        compiler_params=pltpu.CompilerParams(needs_layout_passes=False),
    )
    def gather(table_hbm, idx_hbm, out_hbm, idx_v, out_v, sem):
        wid = lax.axis_index("s") * NC + lax.axis_index("c")
        base = wid * B_PER_W
        pltpu.sync_copy(idx_hbm.at[pl.ds(base, B_PER_W)], idx_v)
        iv = idx_v[...]
        t = lax.iota(jnp.int32, LANES)
        copies = []
        for i in range(B_PER_W):
            row_i = jnp.max(jnp.where(t == i, iv, 0))
            copies.append(pltpu.async_copy(table_hbm.at[row_i], out_v.at[i], sem))
        for c in copies:
            c.wait()
        pltpu.sync_copy(out_v, out_hbm.at[pl.ds(base, B_PER_W)])

    return gather


def _dense_body(e_ref, w1_ref, b1_ref, w2_ref, b2_ref, o_ref, h_ref):
    k = pl.program_id(0)

    @pl.when(k == 0)
    def _():
        h = jnp.dot(e_ref[...], w1_ref[...], preferred_element_type=jnp.float32)
        h_ref[...] = jnp.maximum(h + b1_ref[...], 0.0)

    chunk = jnp.dot(h_ref[...], w2_ref[...], preferred_element_type=jnp.float32)
    chunk = chunk + b2_ref[...]
    col = k * CHUNK + lax.broadcasted_iota(jnp.int32, (1, CHUNK), 1)
    chunk = jnp.where(col < VOCAB, chunk, -1e30)
    o_ref[:, pl.ds(k * CHUNK, CHUNK)] = chunk

    @pl.when(k == NCHUNK - 1)
    def _():
        logits = o_ref[...]
        m = jnp.max(logits)
        lse = m + jnp.log(jnp.sum(jnp.exp(logits - m)))
        o_ref[...] = logits - lse


def _dense_call(e, W1, b1r, W2, b2r, interpret=False):
    return pl.pallas_call(
        _dense_body,
        grid=(NCHUNK,),
        in_specs=[
            pl.BlockSpec((1, CONTEXT * EMBED), lambda k: (0, 0)),
            pl.BlockSpec((CONTEXT * EMBED, HIDDEN), lambda k: (0, 0)),
            pl.BlockSpec((1, HIDDEN), lambda k: (0, 0)),
            pl.BlockSpec((HIDDEN, CHUNK), lambda k: (0, k)),
            pl.BlockSpec((1, CHUNK), lambda k: (0, k)),
        ],
        out_specs=pl.BlockSpec((1, VPAD), lambda k: (0, 0)),
        out_shape=jax.ShapeDtypeStruct((1, VPAD), jnp.float32),
        scratch_shapes=[pltpu.VMEM((1, HIDDEN), jnp.float32)],
        interpret=interpret,
    )(e, W1, b1r, W2, b2r)


def kernel(inputs, embeddings, W1, b1, W2, b2):
    idx = jnp.zeros((B_PAD,), jnp.int32).at[:CONTEXT].set(inputs.astype(jnp.int32))
    rows = _gather_sc()(embeddings, idx)
    e = rows[:CONTEXT].reshape(1, CONTEXT * EMBED)
    out = _dense_call(e, W1, b1.reshape(1, HIDDEN), W2, b2.reshape(1, VOCAB))
    return out[:, :VOCAB]


# speedup vs baseline: 1.4361x; 1.4361x over previous
import functools

import jax
import jax.numpy as jnp
from jax import lax
from jax.experimental import pallas as pl
from jax.experimental.pallas import tpu as pltpu
from jax.experimental.pallas import tpu_sc as plsc

VOCAB = 100000
EMBED = 64
CONTEXT = 200
HIDDEN = 128

CHUNK = 8192
NCHUNK = -(-VOCAB // CHUNK)
VPAD = NCHUNK * CHUNK

NC, NS = 2, 16
NW = NC * NS
LANES = 16
B_PAD = NW * LANES
B_PER_W = B_PAD // NW
SUPER = 12500


@functools.cache
def _gather_sc():
    @functools.partial(
        pl.kernel,
        mesh=plsc.VectorSubcoreMesh(core_axis_name="c", subcore_axis_name="s"),
        out_type=jax.ShapeDtypeStruct((B_PAD, EMBED), jnp.float32),
        scratch_types=[
            pltpu.VMEM((B_PER_W,), jnp.int32),
            pltpu.VMEM((B_PER_W, EMBED), jnp.float32),
            pltpu.SemaphoreType.DMA,
        ],
        compiler_params=pltpu.CompilerParams(needs_layout_passes=False),
    )
    def gather(table_hbm, idx_hbm, out_hbm, idx_v, out_v, sem):
        wid = lax.axis_index("s") * NC + lax.axis_index("c")
        base = wid * B_PER_W
        pltpu.sync_copy(idx_hbm.at[pl.ds(base, B_PER_W)], idx_v)
        iv = idx_v[...]
        t = lax.iota(jnp.int32, LANES)
        copies = []
        for i in range(B_PER_W):
            row_i = jnp.max(jnp.where(t == i, iv, 0))
            copies.append(pltpu.async_copy(table_hbm.at[row_i], out_v.at[i], sem))
        for c in copies:
            c.wait()
        pltpu.sync_copy(out_v, out_hbm.at[pl.ds(base, B_PER_W)])

    return gather


def _dense_body(e_ref, w1_ref, b1_ref, w2t_ref, b2_ref, o_ref, h_ref):
    k = pl.program_id(0)

    @pl.when(k == 0)
    def _():
        h = jnp.dot(e_ref[...], w1_ref[...], preferred_element_type=jnp.float32)
        h_ref[...] = jnp.maximum(h + b1_ref[...], 0.0)

    chunk = lax.dot_general(h_ref[...], w2t_ref[...],
                            (((1,), (1,)), ((), ())),
                            preferred_element_type=jnp.float32)
    chunk = chunk + b2_ref[...]
    col = k * CHUNK + lax.broadcasted_iota(jnp.int32, (1, CHUNK), 1)
    chunk = jnp.where(col < VOCAB, chunk, -1e30)
    o_ref[:, pl.ds(k * CHUNK, CHUNK)] = chunk

    @pl.when(k == NCHUNK - 1)
    def _():
        logits = o_ref[...]
        m = jnp.max(logits)
        lse = m + jnp.log(jnp.sum(jnp.exp(logits - m)))
        o_ref[...] = logits - lse


def _dense_call(e, W1, b1r, W2t, b2r, interpret=False):
    return pl.pallas_call(
        _dense_body,
        grid=(NCHUNK,),
        in_specs=[
            pl.BlockSpec((1, CONTEXT * EMBED), lambda k: (0, 0)),
            pl.BlockSpec((CONTEXT * EMBED, HIDDEN), lambda k: (0, 0)),
            pl.BlockSpec((1, HIDDEN), lambda k: (0, 0)),
            pl.BlockSpec((CHUNK, HIDDEN), lambda k: (k, 0)),
            pl.BlockSpec((1, CHUNK), lambda k: (0, k)),
        ],
        out_specs=pl.BlockSpec((1, VPAD), lambda k: (0, 0)),
        out_shape=jax.ShapeDtypeStruct((1, VPAD), jnp.float32),
        scratch_shapes=[pltpu.VMEM((1, HIDDEN), jnp.float32)],
        interpret=interpret,
    )(e, W1, b1r, W2t, b2r)


def kernel(inputs, embeddings, W1, b1, W2, b2):
    idx = jnp.zeros((B_PAD,), jnp.int32).at[:CONTEXT].set(inputs.astype(jnp.int32))
    rows = _gather_sc()(embeddings, idx)
    e = rows[:CONTEXT].reshape(1, CONTEXT * EMBED)
    out = _dense_call(e, W1, b1.reshape(1, HIDDEN), W2.T, b2.reshape(1, VOCAB))
    return out[:, :VOCAB]


# speedup vs baseline: 2.4383x; 1.6979x over previous
import functools

import jax
import jax.numpy as jnp
from jax import lax
from jax.experimental import pallas as pl
from jax.experimental.pallas import tpu as pltpu
from jax.experimental.pallas import tpu_sc as plsc

VOCAB = 100000
EMBED = 64
CONTEXT = 200
HIDDEN = 128

CHUNK = 8192
NCHUNK = -(-VOCAB // CHUNK)
VPAD = NCHUNK * CHUNK

NC, NS = 2, 16
NW = NC * NS
LANES = 16
B_PAD = NW * LANES
B_PER_W = B_PAD // NW
SUPER = 12500


@functools.cache
def _gather_sc():
    @functools.partial(
        pl.kernel,
        mesh=plsc.VectorSubcoreMesh(core_axis_name="c", subcore_axis_name="s"),
        out_type=jax.ShapeDtypeStruct((B_PAD, EMBED), jnp.float32),
        scratch_types=[
            pltpu.VMEM((B_PER_W,), jnp.int32),
            pltpu.VMEM((8, 8, 8, 128), jnp.float32),
            pltpu.VMEM((B_PER_W, EMBED), jnp.float32),
            pltpu.SemaphoreType.DMA,
        ],
        compiler_params=pltpu.CompilerParams(needs_layout_passes=False),
    )
    def gather(table_hbm, idx_hbm, out_hbm, idx_v, staged, out_v, sem):
        wid = lax.axis_index("s") * NC + lax.axis_index("c")
        base = wid * B_PER_W

        @pl.when(base < CONTEXT)
        def _():
            pltpu.sync_copy(idx_hbm.at[pl.ds(base, B_PER_W)], idx_v)
            iv = idx_v[...]
            t = lax.iota(jnp.int32, LANES)
            for b in range(B_PER_W // 8):
                copies = []
                rms = []
                for s in range(8):
                    row = jnp.max(jnp.where(t == b * 8 + s, iv, 0))
                    rb = lax.shift_right_logical(row, 7)
                    rms.append(lax.bitwise_and(row, 127))
                    copies.append(pltpu.async_copy(
                        table_hbm.at[:, :, pl.ds(rb * 128, 128)], staged.at[s], sem))
                for cpy in copies:
                    cpy.wait()
                for s in range(8):
                    rm = lax.broadcast(rms[s], (LANES,))
                    sv = jnp.full((LANES,), s, jnp.int32)
                    for q in range(EMBED // LANES):
                        jv = t + LANES * q
                        av = lax.shift_right_logical(jv, 3)
                        cv = lax.bitwise_and(jv, 7)
                        v = plsc.load_gather(staged, [sv, av, cv, rm])
                        out_v[b * 8 + s, pl.ds(LANES * q, LANES)] = v
            pltpu.sync_copy(out_v, out_hbm.at[pl.ds(base, B_PER_W)])

    return gather


def _dense_body(e_ref, w1_ref, b1_ref, w2t_ref, b2_ref, o_ref, h_ref):
    k = pl.program_id(0)

    @pl.when(k == 0)
    def _():
        h = jnp.dot(e_ref[...], w1_ref[...], preferred_element_type=jnp.float32)
        h_ref[...] = jnp.maximum(h + b1_ref[...], 0.0)

    chunk = lax.dot_general(h_ref[...], w2t_ref[...],
                            (((1,), (1,)), ((), ())),
                            preferred_element_type=jnp.float32)
    chunk = chunk + b2_ref[...]
    col = k * CHUNK + lax.broadcasted_iota(jnp.int32, (1, CHUNK), 1)
    chunk = jnp.where(col < VOCAB, chunk, -1e30)
    o_ref[:, pl.ds(k * CHUNK, CHUNK)] = chunk

    @pl.when(k == NCHUNK - 1)
    def _():
        logits = o_ref[...]
        m = jnp.max(logits)
        lse = m + jnp.log(jnp.sum(jnp.exp(logits - m)))
        o_ref[...] = logits - lse


def _dense_call(e, W1, b1r, W2t, b2r, interpret=False):
    return pl.pallas_call(
        _dense_body,
        grid=(NCHUNK,),
        in_specs=[
            pl.BlockSpec((1, CONTEXT * EMBED), lambda k: (0, 0)),
            pl.BlockSpec((CONTEXT * EMBED, HIDDEN), lambda k: (0, 0)),
            pl.BlockSpec((1, HIDDEN), lambda k: (0, 0)),
            pl.BlockSpec((CHUNK, HIDDEN), lambda k: (k, 0)),
            pl.BlockSpec((1, CHUNK), lambda k: (0, k)),
        ],
        out_specs=pl.BlockSpec((1, VPAD), lambda k: (0, 0)),
        out_shape=jax.ShapeDtypeStruct((1, VPAD), jnp.float32),
        scratch_shapes=[pltpu.VMEM((1, HIDDEN), jnp.float32)],
        interpret=interpret,
    )(e, W1, b1r, W2t, b2r)


def kernel(inputs, embeddings, W1, b1, W2, b2):
    idx = jnp.zeros((B_PAD,), jnp.int32).at[:CONTEXT].set(inputs.astype(jnp.int32))
    table3 = embeddings.T.reshape(8, 8, VOCAB)
    rows = _gather_sc()(table3, idx)
    e = rows[:CONTEXT].reshape(1, CONTEXT * EMBED)
    out = _dense_call(e, W1, b1.reshape(1, HIDDEN), W2.T, b2.reshape(1, VOCAB))
    return out[:, :VOCAB]


# speedup vs baseline: 2.7219x; 1.1163x over previous
import functools

import jax
import jax.numpy as jnp
from jax import lax
from jax.experimental import pallas as pl
from jax.experimental.pallas import tpu as pltpu
from jax.experimental.pallas import tpu_sc as plsc

VOCAB = 100000
EMBED = 64
CONTEXT = 200
HIDDEN = 128

CHUNK = 8192
NCHUNK = -(-VOCAB // CHUNK)
VPAD = NCHUNK * CHUNK

NC, NS = 2, 16
LANES = 16
B_PER_W = 8
NWORK = CONTEXT // B_PER_W


@functools.cache
def _gather_sc():
    @functools.partial(
        pl.kernel,
        mesh=plsc.VectorSubcoreMesh(core_axis_name="c", subcore_axis_name="s"),
        out_type=jax.ShapeDtypeStruct((1, CONTEXT * EMBED), jnp.float32),
        scratch_types=[
            pltpu.VMEM((LANES,), jnp.int32),
            pltpu.VMEM((B_PER_W, 8, 8, 128), jnp.float32),
            pltpu.VMEM((1, B_PER_W * EMBED), jnp.float32),
            pltpu.SemaphoreType.DMA,
        ],
        compiler_params=pltpu.CompilerParams(needs_layout_passes=False),
    )
    def gather(table_hbm, idx_hbm, out_hbm, idx_v, staged, out_v, sem):
        wid = lax.axis_index("s") * NC + lax.axis_index("c")
        base = wid * B_PER_W

        @pl.when(wid < NWORK)
        def _():
            pltpu.sync_copy(idx_hbm.at[pl.ds(base, B_PER_W)],
                            idx_v.at[pl.ds(0, B_PER_W)])
            iv = idx_v[...]
            t = lax.iota(jnp.int32, LANES)
            copies = []
            rms = []
            for s in range(B_PER_W):
                row = jnp.max(jnp.where(t == s, iv, 0))
                rb = lax.shift_right_logical(row, 7)
                rms.append(lax.bitwise_and(row, 127))
                copies.append(pltpu.async_copy(
                    table_hbm.at[:, :, pl.ds(rb * 128, 128)], staged.at[s], sem))
            for cpy in copies:
                cpy.wait()
            for s in range(B_PER_W):
                rm = lax.broadcast(rms[s], (LANES,))
                sv = jnp.full((LANES,), s, jnp.int32)
                for q in range(EMBED // LANES):
                    jv = t + LANES * q
                    av = lax.shift_right_logical(jv, 3)
                    cv = lax.bitwise_and(jv, 7)
                    v = plsc.load_gather(staged, [sv, av, cv, rm])
                    out_v[0, pl.ds(s * EMBED + LANES * q, LANES)] = v
            pltpu.sync_copy(out_v, out_hbm.at[:, pl.ds(base * EMBED, B_PER_W * EMBED)])

    return gather


def _dense_body(e_ref, w1_ref, b1_ref, w2t_ref, b2_ref, o_ref, h_ref, m_ref, s_ref):
    k = pl.program_id(0)

    @pl.when(k == 0)
    def _():
        h = jnp.dot(e_ref[...], w1_ref[...], preferred_element_type=jnp.float32)
        h_ref[...] = jnp.maximum(h + b1_ref[...][None, :], 0.0)

    chunk = lax.dot_general(h_ref[...], w2t_ref[...],
                            (((1,), (1,)), ((), ())),
                            preferred_element_type=jnp.float32)
    chunk = chunk + b2_ref[...][None, :]
    col = k * CHUNK + lax.broadcasted_iota(jnp.int32, (1, CHUNK), 1)
    chunk = jnp.where(col < VOCAB, chunk, -1e30)
    o_ref[:, pl.ds(k * CHUNK, CHUNK)] = chunk

    @pl.when(k == 0)
    def _():
        m_ref[...] = chunk
        s_ref[...] = jnp.ones_like(chunk)

    @pl.when(k > 0)
    def _():
        m_old = m_ref[...]
        m_new = jnp.maximum(m_old, chunk)
        s_ref[...] = s_ref[...] * jnp.exp(m_old - m_new) + jnp.exp(chunk - m_new)
        m_ref[...] = m_new

    @pl.when(k == NCHUNK - 1)
    def _():
        m_vec = m_ref[...]
        m_glob = jnp.max(m_vec)
        total = jnp.sum(s_ref[...] * jnp.exp(m_vec - m_glob))
        lse = m_glob + jnp.log(total)
        o_ref[...] = o_ref[...] - lse


def _dense_call(e, W1, b1, W2t, b2, interpret=False):
    return pl.pallas_call(
        _dense_body,
        grid=(NCHUNK,),
        in_specs=[
            pl.BlockSpec((1, CONTEXT * EMBED), lambda k: (0, 0)),
            pl.BlockSpec((CONTEXT * EMBED, HIDDEN), lambda k: (0, 0)),
            pl.BlockSpec((HIDDEN,), lambda k: (0,)),
            pl.BlockSpec((CHUNK, HIDDEN), lambda k: (k, 0)),
            pl.BlockSpec((CHUNK,), lambda k: (k,)),
        ],
        out_specs=pl.BlockSpec((1, VPAD), lambda k: (0, 0)),
        out_shape=jax.ShapeDtypeStruct((1, VPAD), jnp.float32),
        scratch_shapes=[
            pltpu.VMEM((1, HIDDEN), jnp.float32),
            pltpu.VMEM((1, CHUNK), jnp.float32),
            pltpu.VMEM((1, CHUNK), jnp.float32),
        ],
        interpret=interpret,
    )(e, W1, b1, W2t, b2)


def kernel(inputs, embeddings, W1, b1, W2, b2):
    table3 = embeddings.T.reshape(8, 8, VOCAB)
    e = _gather_sc()(table3, inputs.astype(jnp.int32))
    out = _dense_call(e, W1, b1, W2.T, b2)
    return out[:, :VOCAB]


# speedup vs baseline: 2.9461x; 1.0824x over previous
import functools

import jax
import jax.numpy as jnp
from jax import lax
from jax.experimental import pallas as pl
from jax.experimental.pallas import tpu as pltpu
from jax.experimental.pallas import tpu_sc as plsc

VOCAB = 100000
EMBED = 64
CONTEXT = 200
HIDDEN = 128

CHUNK = 16384
NCHUNK = -(-VOCAB // CHUNK)
VPAD = NCHUNK * CHUNK

NC, NS = 2, 16
LANES = 16
B_PER_W = 8
NWORK = CONTEXT // B_PER_W


@functools.cache
def _gather_sc():
    @functools.partial(
        pl.kernel,
        mesh=plsc.VectorSubcoreMesh(core_axis_name="c", subcore_axis_name="s"),
        out_type=jax.ShapeDtypeStruct((1, CONTEXT * EMBED), jnp.float32),
        scratch_types=[
            pltpu.VMEM((LANES,), jnp.int32),
            pltpu.VMEM((B_PER_W, 8, 8, 128), jnp.float32),
            pltpu.VMEM((1, B_PER_W * EMBED), jnp.float32),
            pltpu.SemaphoreType.DMA,
        ],
        compiler_params=pltpu.CompilerParams(needs_layout_passes=False),
    )
    def gather(table_hbm, idx_hbm, out_hbm, idx_v, staged, out_v, sem):
        wid = lax.axis_index("s") * NC + lax.axis_index("c")
        base = wid * B_PER_W

        @pl.when(wid < NWORK)
        def _():
            pltpu.sync_copy(idx_hbm.at[pl.ds(base, B_PER_W)],
                            idx_v.at[pl.ds(0, B_PER_W)])
            iv = idx_v[...]
            t = lax.iota(jnp.int32, LANES)
            copies = []
            rms = []
            for s in range(B_PER_W):
                row = jnp.max(jnp.where(t == s, iv, 0))
                rb = lax.shift_right_logical(row, 7)
                rms.append(lax.bitwise_and(row, 127))
                copies.append(pltpu.async_copy(
                    table_hbm.at[:, :, pl.ds(rb * 128, 128)], staged.at[s], sem))
            for cpy in copies:
                cpy.wait()
            for s in range(B_PER_W):
                rm = lax.broadcast(rms[s], (LANES,))
                sv = jnp.full((LANES,), s, jnp.int32)
                for q in range(EMBED // LANES):
                    jv = t + LANES * q
                    av = lax.shift_right_logical(jv, 3)
                    cv = lax.bitwise_and(jv, 7)
                    v = plsc.load_gather(staged, [sv, av, cv, rm])
                    out_v[0, pl.ds(s * EMBED + LANES * q, LANES)] = v
            pltpu.sync_copy(out_v, out_hbm.at[:, pl.ds(base * EMBED, B_PER_W * EMBED)])

    return gather


def _dense_body(e_ref, w1_ref, b1_ref, w2t_ref, b2_ref, o_ref, h_ref, m_ref, s_ref):
    k = pl.program_id(0)

    @pl.when(k == 0)
    def _():
        h = jnp.dot(e_ref[...], w1_ref[...], preferred_element_type=jnp.float32)
        h_ref[...] = jnp.maximum(h + b1_ref[...][None, :], 0.0)

    chunk = lax.dot_general(h_ref[...], w2t_ref[...],
                            (((1,), (1,)), ((), ())),
                            preferred_element_type=jnp.float32)
    chunk = chunk + b2_ref[...][None, :]
    col = k * CHUNK + lax.broadcasted_iota(jnp.int32, (1, CHUNK), 1)
    chunk = jnp.where(col < VOCAB, chunk, -1e30)
    o_ref[:, pl.ds(k * CHUNK, CHUNK)] = chunk

    @pl.when(k == 0)
    def _():
        m_ref[...] = chunk
        s_ref[...] = jnp.ones_like(chunk)

    @pl.when(k > 0)
    def _():
        m_old = m_ref[...]
        m_new = jnp.maximum(m_old, chunk)
        s_ref[...] = s_ref[...] * jnp.exp(m_old - m_new) + jnp.exp(chunk - m_new)
        m_ref[...] = m_new

    @pl.when(k == NCHUNK - 1)
    def _():
        m_vec = m_ref[...]
        m_glob = jnp.max(m_vec)
        total = jnp.sum(s_ref[...] * jnp.exp(m_vec - m_glob))
        lse = m_glob + jnp.log(total)
        o_ref[...] = o_ref[...] - lse


def _dense_call(e, W1, b1, W2t, b2, interpret=False):
    return pl.pallas_call(
        _dense_body,
        grid=(NCHUNK,),
        in_specs=[
            pl.BlockSpec((1, CONTEXT * EMBED), lambda k: (0, 0)),
            pl.BlockSpec((CONTEXT * EMBED, HIDDEN), lambda k: (0, 0)),
            pl.BlockSpec((HIDDEN,), lambda k: (0,)),
            pl.BlockSpec((CHUNK, HIDDEN), lambda k: (k, 0)),
            pl.BlockSpec((CHUNK,), lambda k: (k,)),
        ],
        out_specs=pl.BlockSpec((1, VPAD), lambda k: (0, 0)),
        out_shape=jax.ShapeDtypeStruct((1, VOCAB), jnp.float32),
        scratch_shapes=[
            pltpu.VMEM((1, HIDDEN), jnp.float32),
            pltpu.VMEM((1, CHUNK), jnp.float32),
            pltpu.VMEM((1, CHUNK), jnp.float32),
        ],
        interpret=interpret,
    )(e, W1, b1, W2t, b2)


def kernel(inputs, embeddings, W1, b1, W2, b2):
    table3 = embeddings.T.reshape(8, 8, VOCAB)
    e = _gather_sc()(table3, inputs.astype(jnp.int32))
    return _dense_call(e, W1, b1, W2.T, b2)
